# baseline (speedup 1.0000x reference)
import functools

import jax
import jax.numpy as jnp
from jax import lax
from jax.experimental import pallas as pl
from jax.experimental.pallas import tpu as pltpu
from jax.experimental.pallas import tpu_sc as plsc

_PAIR = 128


def _make_sc_gather(vpairs, n, chunks):
    info = plsc.get_sparse_core_info()
    NC, NS = info.num_cores, info.num_subcores
    NW = NC * NS
    assert n % (8 * NW) == 0
    b_per_w = n // NW
    assert b_per_w % chunks == 0
    c_rows = b_per_w // chunks
    mesh = plsc.VectorSubcoreMesh(core_axis_name="c", subcore_axis_name="s")

    @functools.partial(
        pl.kernel,
        mesh=mesh,
        out_type=jax.ShapeDtypeStruct((n, _PAIR), jnp.float32),
        scratch_types=[
            pltpu.VMEM((b_per_w,), jnp.int32),
            pltpu.VMEM((c_rows, _PAIR), jnp.float32),
            pltpu.SemaphoreType.DMA,
        ],
        compiler_params=pltpu.CompilerParams(use_tc_tiling_on_sc=False),
    )
    def gather(table_hbm, idx_hbm, out_hbm, idx_v, rows_v, sem):
        wid = lax.axis_index("s") * NC + lax.axis_index("c")
        base = wid * b_per_w
        pltpu.sync_copy(idx_hbm.at[pl.ds(base, b_per_w)], idx_v)
        for c in range(chunks):
            pltpu.async_copy(
                table_hbm.at[idx_v.at[pl.ds(c * c_rows, c_rows)]], rows_v, sem
            ).wait()
            pltpu.sync_copy(
                rows_v, out_hbm.at[pl.ds(base + c * c_rows, c_rows)]
            )

    return gather


def _proj_body(x_ref, p_ref, w_ref, b_ref, o_ref):
    x = x_ref[...]
    half = jax.lax.broadcasted_iota(jnp.int32, x.shape, 1) // 64
    xsel = jnp.where(half == p_ref[...], x, 0.0)
    o_ref[...] = (
        jnp.dot(xsel, w_ref[...], preferred_element_type=jnp.float32)
        + b_ref[...]
    )


def _project(emb, parity, W2, b, block_rows=256):
    n = emb.shape[0]
    h = W2.shape[1]
    return pl.pallas_call(
        _proj_body,
        grid=(n // block_rows,),
        in_specs=[
            pl.BlockSpec((block_rows, _PAIR), lambda i: (i, 0)),
            pl.BlockSpec((block_rows, 1), lambda i: (i, 0)),
            pl.BlockSpec((_PAIR, h), lambda i: (0, 0)),
            pl.BlockSpec((1, h), lambda i: (0, 0)),
        ],
        out_specs=pl.BlockSpec((block_rows, h), lambda i: (i, 0)),
        out_shape=jax.ShapeDtypeStruct((n, h), jnp.float32),
        compiler_params=pltpu.CompilerParams(
            dimension_semantics=("arbitrary",)
        ),
    )(emb, parity, W2, b.reshape(1, h))


def kernel(text_ids, table, W, b):
    batch, seq = text_ids.shape
    vocab, d = table.shape
    idx = text_ids.reshape(-1)
    n = idx.shape[0]
    table2 = table.reshape(vocab // 2, _PAIR)
    idx2 = idx // 2
    parity = (idx % 2).reshape(n, 1)
    gather = _make_sc_gather(vocab // 2, n, chunks=2)
    emb2 = gather(table2, idx2)
    W2 = jnp.concatenate([W, W], axis=0)
    out = _project(emb2, parity, W2, b)
    return out.reshape(batch, seq, W.shape[1])


# speedup vs baseline: 1.1685x; 1.1685x over previous
import functools

import jax
import jax.numpy as jnp
from jax import lax
from jax.experimental import pallas as pl
from jax.experimental.pallas import tpu as pltpu
from jax.experimental.pallas import tpu_sc as plsc


def _make_sc_gather(vocab, d, n):
    info = plsc.get_sparse_core_info()
    NC, NS = info.num_cores, info.num_subcores
    NW = NC * NS
    assert n % (8 * NW) == 0
    b_per_w = n // NW
    mesh = plsc.VectorSubcoreMesh(core_axis_name="c", subcore_axis_name="s")

    @functools.partial(
        pl.kernel,
        mesh=mesh,
        out_type=jax.ShapeDtypeStruct((n, d), jnp.bfloat16),
        scratch_types=[
            pltpu.VMEM((b_per_w,), jnp.int32),
            pltpu.VMEM((b_per_w, d), jnp.bfloat16),
            pltpu.SemaphoreType.DMA,
        ],
        compiler_params=pltpu.CompilerParams(use_tc_tiling_on_sc=False),
    )
    def gather(tab_hbm, idx_hbm, out_hbm, idx_v, rows_v, sem):
        wid = lax.axis_index("s") * NC + lax.axis_index("c")
        base = wid * b_per_w
        pltpu.sync_copy(idx_hbm.at[pl.ds(base, b_per_w)], idx_v)
        pltpu.async_copy(tab_hbm.at[idx_v], rows_v, sem).wait()
        pltpu.sync_copy(rows_v, out_hbm.at[pl.ds(base, b_per_w)])

    return gather


def _proj_body(x_ref, w_ref, b_ref, o_ref):
    o_ref[...] = (
        jnp.dot(
            x_ref[...].astype(jnp.float32),
            w_ref[...],
            preferred_element_type=jnp.float32,
        )
        + b_ref[...]
    )


def _project(emb, W, b, block_rows=512):
    n, d = emb.shape
    h = W.shape[1]
    return pl.pallas_call(
        _proj_body,
        grid=(n // block_rows,),
        in_specs=[
            pl.BlockSpec((block_rows, d), lambda i: (i, 0)),
            pl.BlockSpec((d, h), lambda i: (0, 0)),
            pl.BlockSpec((1, h), lambda i: (0, 0)),
        ],
        out_specs=pl.BlockSpec((block_rows, h), lambda i: (i, 0)),
        out_shape=jax.ShapeDtypeStruct((n, h), jnp.float32),
        compiler_params=pltpu.CompilerParams(
            dimension_semantics=("arbitrary",)
        ),
    )(emb, W, b.reshape(1, h))


def kernel(text_ids, table, W, b):
    batch, seq = text_ids.shape
    vocab, d = table.shape
    h = W.shape[1]
    idx = text_ids.T.reshape(-1)
    n = idx.shape[0]
    tb = table.astype(jnp.bfloat16)
    gather = _make_sc_gather(vocab, d, n)
    emb = gather(tb, idx)
    out = _project(emb, W, b)
    return out.reshape(seq, batch, h).transpose(1, 0, 2)


# speedup vs baseline: 2.8932x; 2.4760x over previous
import functools

import jax
import jax.numpy as jnp
from jax import lax
from jax.experimental import pallas as pl
from jax.experimental.pallas import tpu as pltpu
from jax.experimental.pallas import tpu_sc as plsc

_PAIR = 128


_HALF = 524288


def _transpose_body(x1_ref, x2_ref, o_ref):
    o_ref[...] = jnp.concatenate([x1_ref[...].T, x2_ref[...].T], axis=1)


def _pair_table(tabT, block_k=8192):
    d, v = tabT.shape
    hblk = _HALF // block_k
    last = (v - 1) // block_k
    return pl.pallas_call(
        _transpose_body,
        grid=(hblk,),
        in_specs=[
            pl.BlockSpec((d, block_k), lambda i: (0, i)),
            pl.BlockSpec(
                (d, block_k), lambda i: (0, jnp.minimum(i + hblk, last))
            ),
        ],
        out_specs=pl.BlockSpec((block_k, _PAIR), lambda i: (i, 0)),
        out_shape=jax.ShapeDtypeStruct((_HALF, _PAIR), jnp.float32),
        compiler_params=pltpu.CompilerParams(
            dimension_semantics=("arbitrary",)
        ),
    )(tabT, tabT)


def _make_sc_gather(vpairs, n, chunks):
    info = plsc.get_sparse_core_info()
    NC, NS = info.num_cores, info.num_subcores
    NW = NC * NS
    assert n % (8 * NW) == 0
    b_per_w = n // NW
    assert b_per_w % chunks == 0
    c_rows = b_per_w // chunks
    mesh = plsc.VectorSubcoreMesh(core_axis_name="c", subcore_axis_name="s")

    @functools.partial(
        pl.kernel,
        mesh=mesh,
        out_type=jax.ShapeDtypeStruct((n, _PAIR), jnp.float32),
        scratch_types=[
            pltpu.VMEM((b_per_w,), jnp.int32),
            pltpu.VMEM((c_rows, _PAIR), jnp.float32),
            pltpu.SemaphoreType.DMA,
        ],
        compiler_params=pltpu.CompilerParams(use_tc_tiling_on_sc=False),
    )
    def gather(table_hbm, idx_hbm, out_hbm, idx_v, rows_v, sem):
        wid = lax.axis_index("s") * NC + lax.axis_index("c")
        base = wid * b_per_w
        pltpu.sync_copy(idx_hbm.at[pl.ds(base, b_per_w)], idx_v)
        for c in range(chunks):
            pltpu.async_copy(
                table_hbm.at[idx_v.at[pl.ds(c * c_rows, c_rows)]], rows_v, sem
            ).wait()
            pltpu.sync_copy(
                rows_v, out_hbm.at[pl.ds(base + c * c_rows, c_rows)]
            )

    return gather


def _proj_body(x_ref, p_ref, w_ref, b_ref, o_ref):
    x = x_ref[...]
    half = jax.lax.broadcasted_iota(jnp.int32, x.shape, 1) // 64
    xsel = jnp.where(half == p_ref[...], x, 0.0)
    o_ref[...] = (
        jnp.dot(xsel, w_ref[...], preferred_element_type=jnp.float32)
        + b_ref[...]
    )


def _project(emb, parity, W2, b, block_rows=512):
    n = emb.shape[0]
    h = W2.shape[1]
    return pl.pallas_call(
        _proj_body,
        grid=(n // block_rows,),
        in_specs=[
            pl.BlockSpec((block_rows, _PAIR), lambda i: (i, 0)),
            pl.BlockSpec((block_rows, 1), lambda i: (i, 0)),
            pl.BlockSpec((_PAIR, h), lambda i: (0, 0)),
            pl.BlockSpec((1, h), lambda i: (0, 0)),
        ],
        out_specs=pl.BlockSpec((block_rows, h), lambda i: (i, 0)),
        out_shape=jax.ShapeDtypeStruct((n, h), jnp.float32),
        compiler_params=pltpu.CompilerParams(
            dimension_semantics=("arbitrary",)
        ),
    )(emb, parity, W2, b.reshape(1, h))


def kernel(text_ids, table, W, b):
    batch, seq = text_ids.shape
    vocab, d = table.shape
    h = W.shape[1]
    idx = text_ids.T.reshape(-1)
    n = idx.shape[0]
    table2 = _pair_table(table.T)
    idx2 = idx & (_HALF - 1)
    parity = (idx >> 19).reshape(n, 1)
    gather = _make_sc_gather(_HALF, n, chunks=2)
    emb2 = gather(table2, idx2)
    W2 = jnp.concatenate([W, W], axis=0)
    out = _project(emb2, parity, W2, b)
    return out.reshape(seq, batch, h).transpose(1, 0, 2)


# speedup vs baseline: 3.2486x; 1.1228x over previous
import functools

import jax
import jax.numpy as jnp
from jax import lax
from jax.experimental import pallas as pl
from jax.experimental.pallas import tpu as pltpu
from jax.experimental.pallas import tpu_sc as plsc

_PAIR = 128


_HALF = 524288


def _transpose_body(x1_ref, x2_ref, o_ref):
    o_ref[...] = jnp.concatenate([x1_ref[...].T, x2_ref[...].T], axis=1)


def _pair_table(tabT, block_k=16384):
    d, v = tabT.shape
    hblk = _HALF // block_k
    last = (v - 1) // block_k
    return pl.pallas_call(
        _transpose_body,
        grid=(hblk,),
        in_specs=[
            pl.BlockSpec((d, block_k), lambda i: (0, i)),
            pl.BlockSpec(
                (d, block_k), lambda i: (0, jnp.minimum(i + hblk, last))
            ),
        ],
        out_specs=pl.BlockSpec((block_k, _PAIR), lambda i: (i, 0)),
        out_shape=jax.ShapeDtypeStruct((_HALF, _PAIR), jnp.float32),
        compiler_params=pltpu.CompilerParams(
            dimension_semantics=("parallel",)
        ),
    )(tabT, tabT)


def _make_sc_gather(vpairs, n, chunks):
    info = plsc.get_sparse_core_info()
    NC, NS = info.num_cores, info.num_subcores
    NW = NC * NS
    assert n % (8 * NW) == 0
    b_per_w = n // NW
    assert b_per_w % chunks == 0
    c_rows = b_per_w // chunks
    mesh = plsc.VectorSubcoreMesh(core_axis_name="c", subcore_axis_name="s")

    @functools.partial(
        pl.kernel,
        mesh=mesh,
        out_type=jax.ShapeDtypeStruct((n, _PAIR), jnp.float32),
        scratch_types=[
            pltpu.VMEM((b_per_w,), jnp.int32),
            pltpu.VMEM((c_rows, _PAIR), jnp.float32),
            pltpu.SemaphoreType.DMA,
        ],
        compiler_params=pltpu.CompilerParams(use_tc_tiling_on_sc=False),
    )
    def gather(table_hbm, idx_hbm, out_hbm, idx_v, rows_v, sem):
        wid = lax.axis_index("s") * NC + lax.axis_index("c")
        base = wid * b_per_w
        pltpu.sync_copy(idx_hbm.at[pl.ds(base, b_per_w)], idx_v)
        for c in range(chunks):
            pltpu.async_copy(
                table_hbm.at[idx_v.at[pl.ds(c * c_rows, c_rows)]], rows_v, sem
            ).wait()
            pltpu.sync_copy(
                rows_v, out_hbm.at[pl.ds(base + c * c_rows, c_rows)]
            )

    return gather


def _proj_body(x_ref, p_ref, w_ref, b_ref, o_ref):
    x = x_ref[...]
    half = jax.lax.broadcasted_iota(jnp.int32, x.shape, 1) // 64
    xsel = jnp.where(half == p_ref[...], x, 0.0).astype(jnp.bfloat16)
    o_ref[...] = (
        jnp.dot(xsel, w_ref[...], preferred_element_type=jnp.float32)
        + b_ref[...]
    )


def _project(emb, parity, W2, b, block_rows=1024):
    n = emb.shape[0]
    h = W2.shape[1]
    return pl.pallas_call(
        _proj_body,
        grid=(n // block_rows,),
        in_specs=[
            pl.BlockSpec((block_rows, _PAIR), lambda i: (i, 0)),
            pl.BlockSpec((block_rows, 1), lambda i: (i, 0)),
            pl.BlockSpec((_PAIR, h), lambda i: (0, 0)),
            pl.BlockSpec((1, h), lambda i: (0, 0)),
        ],
        out_specs=pl.BlockSpec((block_rows, h), lambda i: (i, 0)),
        out_shape=jax.ShapeDtypeStruct((n, h), jnp.float32),
        compiler_params=pltpu.CompilerParams(
            dimension_semantics=("parallel",)
        ),
    )(emb, parity, W2, b.reshape(1, h))


def kernel(text_ids, table, W, b):
    batch, seq = text_ids.shape
    vocab, d = table.shape
    h = W.shape[1]
    idx = text_ids.T.reshape(-1)
    n = idx.shape[0]
    table2 = _pair_table(table.T)
    idx2 = idx & (_HALF - 1)
    parity = (idx >> 19).reshape(n, 1)
    gather = _make_sc_gather(_HALF, n, chunks=2)
    emb2 = gather(table2, idx2)
    W2 = jnp.concatenate([W, W], axis=0).astype(jnp.bfloat16)
    out = _project(emb2, parity, W2, b)
    return out.reshape(seq, batch, h).transpose(1, 0, 2)


# speedup vs baseline: 4.0513x; 1.2471x over previous
import functools

import jax
import jax.numpy as jnp
from jax import lax
from jax.experimental import pallas as pl
from jax.experimental.pallas import tpu as pltpu
from jax.experimental.pallas import tpu_sc as plsc

_PAIR = 128
_QUART = 262144


def _pack2(a_ref, b_ref):
    au = lax.bitcast_convert_type(
        a_ref[...].T.astype(jnp.bfloat16), jnp.uint16
    ).astype(jnp.uint32)
    bu = lax.bitcast_convert_type(
        b_ref[...].T.astype(jnp.bfloat16), jnp.uint16
    ).astype(jnp.uint32)
    return lax.bitcast_convert_type(au | (bu << 16), jnp.float32)


def _transpose_body(x1_ref, x2_ref, x3_ref, x4_ref, o_ref):
    o_ref[...] = jnp.concatenate(
        [_pack2(x1_ref, x2_ref), _pack2(x3_ref, x4_ref)], axis=1
    )


def _pair_table(tabT, block_k=8192):
    d, v = tabT.shape
    hblk = _QUART // block_k
    last = (v - 1) // block_k

    def mk(s):
        return pl.BlockSpec(
            (d, block_k), lambda i: (0, jnp.minimum(i + s * hblk, last))
        )

    return pl.pallas_call(
        _transpose_body,
        grid=(hblk,),
        in_specs=[mk(0), mk(1), mk(2), mk(3)],
        out_specs=pl.BlockSpec((block_k, _PAIR), lambda i: (i, 0)),
        out_shape=jax.ShapeDtypeStruct((_QUART, _PAIR), jnp.float32),
        compiler_params=pltpu.CompilerParams(
            dimension_semantics=("parallel",)
        ),
    )(tabT, tabT, tabT, tabT)


def _make_sc_gather(vpairs, n, chunks):
    info = plsc.get_sparse_core_info()
    NC, NS = info.num_cores, info.num_subcores
    NW = NC * NS
    assert n % (8 * NW) == 0
    b_per_w = n // NW
    assert b_per_w % chunks == 0
    c_rows = b_per_w // chunks
    mesh = plsc.VectorSubcoreMesh(core_axis_name="c", subcore_axis_name="s")

    @functools.partial(
        pl.kernel,
        mesh=mesh,
        out_type=jax.ShapeDtypeStruct((n, _PAIR), jnp.float32),
        scratch_types=[
            pltpu.VMEM((b_per_w,), jnp.int32),
            pltpu.VMEM((c_rows, _PAIR), jnp.float32),
            pltpu.SemaphoreType.DMA,
        ],
        compiler_params=pltpu.CompilerParams(use_tc_tiling_on_sc=False),
    )
    def gather(table_hbm, idx_hbm, out_hbm, idx_v, rows_v, sem):
        wid = lax.axis_index("s") * NC + lax.axis_index("c")
        base = wid * b_per_w
        pltpu.sync_copy(idx_hbm.at[pl.ds(base, b_per_w)], idx_v)
        for c in range(chunks):
            pltpu.async_copy(
                table_hbm.at[idx_v.at[pl.ds(c * c_rows, c_rows)]], rows_v, sem
            ).wait()
            pltpu.sync_copy(
                rows_v, out_hbm.at[pl.ds(base + c * c_rows, c_rows)]
            )

    return gather


def _proj_body(x_ref, p_ref, w_ref, b_ref, o_ref):
    u = lax.bitcast_convert_type(x_ref[...], jnp.uint32)
    lo = lax.bitcast_convert_type(u.astype(jnp.uint16), jnp.bfloat16)
    hi = lax.bitcast_convert_type(
        (u >> 16).astype(jnp.uint16), jnp.bfloat16
    )
    v = p_ref[...]
    xh = jnp.where((v >> 18) % 2 == 1, hi, lo)
    half = jax.lax.broadcasted_iota(jnp.int32, u.shape, 1) // 64
    xsel = jnp.where(half == (v >> 19), xh, jnp.bfloat16(0.0))
    o_ref[...] = (
        jnp.dot(xsel, w_ref[...], preferred_element_type=jnp.float32)
        + b_ref[...]
    )


def _project(emb, parity, W2, b, block_rows=1024):
    n = emb.shape[0]
    h = W2.shape[1]
    return pl.pallas_call(
        _proj_body,
        grid=(n // block_rows,),
        in_specs=[
            pl.BlockSpec((block_rows, _PAIR), lambda i: (i, 0)),
            pl.BlockSpec((block_rows, 1), lambda i: (i, 0)),
            pl.BlockSpec((_PAIR, h), lambda i: (0, 0)),
            pl.BlockSpec((1, h), lambda i: (0, 0)),
        ],
        out_specs=pl.BlockSpec((block_rows, h), lambda i: (i, 0)),
        out_shape=jax.ShapeDtypeStruct((n, h), jnp.float32),
        compiler_params=pltpu.CompilerParams(
            dimension_semantics=("parallel",)
        ),
    )(emb, parity, W2, b.reshape(1, h))


def kernel(text_ids, table, W, b):
    batch, seq = text_ids.shape
    vocab, d = table.shape
    h = W.shape[1]
    idx = text_ids.T.reshape(-1)
    n = idx.shape[0]
    table2 = _pair_table(table.T)
    idx2 = idx & (_QUART - 1)
    parity = idx.reshape(n, 1)
    gather = _make_sc_gather(_QUART, n, chunks=2)
    emb2 = gather(table2, idx2)
    W2 = jnp.concatenate([W, W], axis=0).astype(jnp.bfloat16)
    out = _project(emb2, parity, W2, b)
    return out.reshape(seq, batch, h).transpose(1, 0, 2)


# speedup vs baseline: 4.2071x; 1.0385x over previous
import functools

import jax
import jax.numpy as jnp
from jax import lax
from jax.experimental import pallas as pl
from jax.experimental.pallas import tpu as pltpu
from jax.experimental.pallas import tpu_sc as plsc

_PAIR = 128
_QUART = 262144


def _pack2(a_ref, b_ref):
    au = lax.bitcast_convert_type(
        a_ref[...].T.astype(jnp.bfloat16), jnp.uint16
    ).astype(jnp.uint32)
    bu = lax.bitcast_convert_type(
        b_ref[...].T.astype(jnp.bfloat16), jnp.uint16
    ).astype(jnp.uint32)
    return lax.bitcast_convert_type(au | (bu << 16), jnp.float32)


def _transpose_body(x1_ref, x2_ref, x3_ref, x4_ref, o_ref):
    o_ref[...] = jnp.concatenate(
        [_pack2(x1_ref, x2_ref), _pack2(x3_ref, x4_ref)], axis=1
    )


def _pair_table(tabT, block_k=8192):
    d, v = tabT.shape
    hblk = _QUART // block_k
    last = (v - 1) // block_k

    def mk(s):
        return pl.BlockSpec(
            (d, block_k), lambda i: (0, jnp.minimum(i + s * hblk, last))
        )

    return pl.pallas_call(
        _transpose_body,
        grid=(hblk,),
        in_specs=[mk(0), mk(1), mk(2), mk(3)],
        out_specs=pl.BlockSpec((block_k, _PAIR), lambda i: (i, 0)),
        out_shape=jax.ShapeDtypeStruct((_QUART, _PAIR), jnp.float32),
        compiler_params=pltpu.CompilerParams(
            dimension_semantics=("parallel",)
        ),
    )(tabT, tabT, tabT, tabT)


def _make_sc_gather(vpairs, n, chunks):
    info = plsc.get_sparse_core_info()
    NC, NS = info.num_cores, info.num_subcores
    NW = NC * NS
    assert n % (8 * NW) == 0
    b_per_w = n // NW
    assert b_per_w % chunks == 0
    c_rows = b_per_w // chunks
    mesh = plsc.VectorSubcoreMesh(core_axis_name="c", subcore_axis_name="s")

    @functools.partial(
        pl.kernel,
        mesh=mesh,
        out_type=jax.ShapeDtypeStruct((n, _PAIR), jnp.float32),
        scratch_types=[
            pltpu.VMEM((b_per_w,), jnp.int32),
            pltpu.VMEM((c_rows, _PAIR), jnp.float32),
            pltpu.SemaphoreType.DMA,
        ],
        compiler_params=pltpu.CompilerParams(use_tc_tiling_on_sc=False),
    )
    def gather(table_hbm, idx_hbm, out_hbm, idx_v, rows_v, sem):
        wid = lax.axis_index("s") * NC + lax.axis_index("c")
        base = wid * b_per_w
        pltpu.sync_copy(idx_hbm.at[pl.ds(base, b_per_w)], idx_v)
        for c in range(chunks):
            pltpu.async_copy(
                table_hbm.at[idx_v.at[pl.ds(c * c_rows, c_rows)]], rows_v, sem
            ).wait()
            pltpu.sync_copy(
                rows_v, out_hbm.at[pl.ds(base + c * c_rows, c_rows)]
            )

    return gather


def _proj_body(x_ref, p_ref, w_ref, b_ref, o_ref):
    u = lax.bitcast_convert_type(x_ref[...], jnp.uint32)
    lo = lax.bitcast_convert_type(u.astype(jnp.uint16), jnp.bfloat16)
    hi = lax.bitcast_convert_type(
        (u >> 16).astype(jnp.uint16), jnp.bfloat16
    )
    v = p_ref[...]
    xh = jnp.where((v >> 18) % 2 == 1, hi, lo)
    half = jax.lax.broadcasted_iota(jnp.int32, u.shape, 1) // 64
    xsel = jnp.where(half == (v >> 19), xh, jnp.bfloat16(0.0))
    o_ref[...] = (
        jnp.dot(xsel, w_ref[...], preferred_element_type=jnp.float32)
        + b_ref[...]
    )


def _project(emb, parity, W2, b, block_rows=2048):
    n = emb.shape[0]
    h = W2.shape[1]
    return pl.pallas_call(
        _proj_body,
        grid=(n // block_rows,),
        in_specs=[
            pl.BlockSpec((block_rows, _PAIR), lambda i: (i, 0)),
            pl.BlockSpec((block_rows, 1), lambda i: (i, 0)),
            pl.BlockSpec((_PAIR, h), lambda i: (0, 0)),
            pl.BlockSpec((1, h), lambda i: (0, 0)),
        ],
        out_specs=pl.BlockSpec((block_rows, h), lambda i: (i, 0)),
        out_shape=jax.ShapeDtypeStruct((n, h), jnp.float32),
        compiler_params=pltpu.CompilerParams(
            dimension_semantics=("parallel",)
        ),
    )(emb, parity, W2, b.reshape(1, h))


def kernel(text_ids, table, W, b):
    batch, seq = text_ids.shape
    vocab, d = table.shape
    h = W.shape[1]
    idx = text_ids.T.reshape(-1)
    n = idx.shape[0]
    table2 = _pair_table(table.T)
    idx2 = idx & (_QUART - 1)
    parity = idx.reshape(n, 1)
    gather = _make_sc_gather(_QUART, n, chunks=2)
    emb2 = gather(table2, idx2)
    W2 = jnp.concatenate([W, W], axis=0).astype(jnp.bfloat16)
    out = _project(emb2, parity, W2, b)
    return out.reshape(seq, batch, h).transpose(1, 0, 2)


# speedup vs baseline: 4.2399x; 1.0078x over previous
import functools

import jax
import jax.numpy as jnp
from jax import lax
from jax.experimental import pallas as pl
from jax.experimental.pallas import tpu as pltpu
from jax.experimental.pallas import tpu_sc as plsc

_PAIR = 128
_QUART = 262144


def _pack2(a_ref, b_ref):
    au = lax.bitcast_convert_type(
        a_ref[...].T.astype(jnp.bfloat16), jnp.uint16
    ).astype(jnp.uint32)
    bu = lax.bitcast_convert_type(
        b_ref[...].T.astype(jnp.bfloat16), jnp.uint16
    ).astype(jnp.uint32)
    return lax.bitcast_convert_type(au | (bu << 16), jnp.float32)


def _transpose_body(x1_ref, x2_ref, x3_ref, x4_ref, o_ref):
    o_ref[...] = jnp.concatenate(
        [_pack2(x1_ref, x2_ref), _pack2(x3_ref, x4_ref)], axis=1
    )


def _pair_table(tabT, block_k=8192):
    d, v = tabT.shape
    hblk = _QUART // block_k
    last = (v - 1) // block_k

    def mk(s):
        return pl.BlockSpec(
            (d, block_k), lambda i: (0, jnp.minimum(i + s * hblk, last))
        )

    return pl.pallas_call(
        _transpose_body,
        grid=(hblk,),
        in_specs=[mk(0), mk(1), mk(2), mk(3)],
        out_specs=pl.BlockSpec((block_k, _PAIR), lambda i: (i, 0)),
        out_shape=jax.ShapeDtypeStruct((_QUART, _PAIR), jnp.float32),
        compiler_params=pltpu.CompilerParams(
            dimension_semantics=("parallel",)
        ),
    )(tabT, tabT, tabT, tabT)


def _make_sc_gather(vpairs, n, chunks):
    info = plsc.get_sparse_core_info()
    NC, NS = info.num_cores, info.num_subcores
    NW = NC * NS
    assert n % (8 * NW) == 0
    b_per_w = n // NW
    assert b_per_w % chunks == 0
    c_rows = b_per_w // chunks
    mesh = plsc.VectorSubcoreMesh(core_axis_name="c", subcore_axis_name="s")

    @functools.partial(
        pl.kernel,
        mesh=mesh,
        out_type=jax.ShapeDtypeStruct((n, _PAIR), jnp.float32),
        scratch_types=[
            pltpu.VMEM((b_per_w,), jnp.int32),
            pltpu.VMEM((c_rows, _PAIR), jnp.float32),
            pltpu.VMEM((c_rows, _PAIR), jnp.float32),
            pltpu.SemaphoreType.DMA,
            pltpu.SemaphoreType.DMA,
        ],
        compiler_params=pltpu.CompilerParams(use_tc_tiling_on_sc=False),
    )
    def gather(table_hbm, idx_hbm, out_hbm, idx_v, rv0, rv1, gsem, wsem):
        wid = lax.axis_index("s") * NC + lax.axis_index("c")
        base = wid * b_per_w
        bufs = [rv0, rv1]
        pltpu.sync_copy(idx_hbm.at[pl.ds(base, b_per_w)], idx_v)

        def g(c):
            return pltpu.make_async_copy(
                table_hbm.at[idx_v.at[pl.ds(c * c_rows, c_rows)]],
                bufs[c % 2],
                gsem,
            )

        def w(c):
            return pltpu.make_async_copy(
                bufs[c % 2], out_hbm.at[pl.ds(base + c * c_rows, c_rows)],
                wsem,
            )

        g(0).start()
        for c in range(chunks):
            g(c).wait()
            w(c).start()
            if c + 1 < chunks:
                if c >= 1:
                    w(c - 1).wait()
                g(c + 1).start()
        w(chunks - 2).wait()
        w(chunks - 1).wait()

    return gather


def _proj_body(x_ref, p_ref, w_ref, b_ref, o_ref):
    u = lax.bitcast_convert_type(x_ref[...], jnp.uint32)
    lo = lax.bitcast_convert_type(u.astype(jnp.uint16), jnp.bfloat16)
    hi = lax.bitcast_convert_type(
        (u >> 16).astype(jnp.uint16), jnp.bfloat16
    )
    v = p_ref[...]
    xh = jnp.where((v >> 18) % 2 == 1, hi, lo)
    half = jax.lax.broadcasted_iota(jnp.int32, u.shape, 1) // 64
    xsel = jnp.where(half == (v >> 19), xh, jnp.bfloat16(0.0))
    o_ref[...] = (
        jnp.dot(xsel, w_ref[...], preferred_element_type=jnp.float32)
        + b_ref[...]
    )


def _project(emb, parity, W2, b, block_rows=2048):
    n = emb.shape[0]
    h = W2.shape[1]
    return pl.pallas_call(
        _proj_body,
        grid=(n // block_rows,),
        in_specs=[
            pl.BlockSpec((block_rows, _PAIR), lambda i: (i, 0)),
            pl.BlockSpec((block_rows, 1), lambda i: (i, 0)),
            pl.BlockSpec((_PAIR, h), lambda i: (0, 0)),
            pl.BlockSpec((1, h), lambda i: (0, 0)),
        ],
        out_specs=pl.BlockSpec((block_rows, h), lambda i: (i, 0)),
        out_shape=jax.ShapeDtypeStruct((n, h), jnp.float32),
        compiler_params=pltpu.CompilerParams(
            dimension_semantics=("parallel",)
        ),
    )(emb, parity, W2, b.reshape(1, h))


def kernel(text_ids, table, W, b):
    batch, seq = text_ids.shape
    vocab, d = table.shape
    h = W.shape[1]
    idx = text_ids.T.reshape(-1)
    n = idx.shape[0]
    table2 = _pair_table(table.T)
    idx2 = idx & (_QUART - 1)
    parity = idx.reshape(n, 1)
    gather = _make_sc_gather(_QUART, n, chunks=4)
    emb2 = gather(table2, idx2)
    W2 = jnp.concatenate([W, W], axis=0).astype(jnp.bfloat16)
    out = _project(emb2, parity, W2, b)
    return out.reshape(seq, batch, h).transpose(1, 0, 2)


# speedup vs baseline: 4.2554x; 1.0037x over previous
import functools

import jax
import jax.numpy as jnp
from jax import lax
from jax.experimental import pallas as pl
from jax.experimental.pallas import tpu as pltpu
from jax.experimental.pallas import tpu_sc as plsc

_PAIR = 128
_QUART = 262144


def _pack2(a_ref, b_ref):
    au = lax.bitcast_convert_type(
        a_ref[...].T.astype(jnp.bfloat16), jnp.uint16
    ).astype(jnp.uint32)
    bu = lax.bitcast_convert_type(
        b_ref[...].T.astype(jnp.bfloat16), jnp.uint16
    ).astype(jnp.uint32)
    return lax.bitcast_convert_type(au | (bu << 16), jnp.float32)


def _transpose_body(x1_ref, x2_ref, x3_ref, x4_ref, o_ref):
    o_ref[...] = jnp.concatenate(
        [_pack2(x1_ref, x2_ref), _pack2(x3_ref, x4_ref)], axis=1
    )


def _pair_table(tabT, block_k=8192):
    d, v = tabT.shape
    hblk = _QUART // block_k
    last = (v - 1) // block_k

    def mk(s):
        return pl.BlockSpec(
            (d, block_k), lambda i: (0, jnp.minimum(i + s * hblk, last))
        )

    return pl.pallas_call(
        _transpose_body,
        grid=(hblk,),
        in_specs=[mk(0), mk(1), mk(2), mk(3)],
        out_specs=pl.BlockSpec((block_k, _PAIR), lambda i: (i, 0)),
        out_shape=jax.ShapeDtypeStruct((_QUART, _PAIR), jnp.float32),
        compiler_params=pltpu.CompilerParams(
            dimension_semantics=("parallel",)
        ),
    )(tabT, tabT, tabT, tabT)


def _make_sc_gather(vpairs, n, chunks):
    info = plsc.get_sparse_core_info()
    NC, NS = info.num_cores, info.num_subcores
    NW = NC * NS
    assert n % (8 * NW) == 0
    b_per_w = n // NW
    assert b_per_w % chunks == 0
    c_rows = b_per_w // chunks
    mesh = plsc.VectorSubcoreMesh(core_axis_name="c", subcore_axis_name="s")

    @functools.partial(
        pl.kernel,
        mesh=mesh,
        out_type=jax.ShapeDtypeStruct((n, _PAIR), jnp.float32),
        scratch_types=[
            pltpu.VMEM((b_per_w,), jnp.int32),
            pltpu.VMEM((c_rows, _PAIR), jnp.float32),
            pltpu.VMEM((c_rows, _PAIR), jnp.float32),
            pltpu.SemaphoreType.DMA,
            pltpu.SemaphoreType.DMA,
        ],
        compiler_params=pltpu.CompilerParams(use_tc_tiling_on_sc=False),
    )
    def gather(table_hbm, idx_hbm, out_hbm, idx_v, rv0, rv1, gsem, wsem):
        wid = lax.axis_index("s") * NC + lax.axis_index("c")
        base = wid * b_per_w
        bufs = [rv0, rv1]
        pltpu.sync_copy(idx_hbm.at[pl.ds(base, b_per_w)], idx_v)

        def g(c):
            return pltpu.make_async_copy(
                table_hbm.at[idx_v.at[pl.ds(c * c_rows, c_rows)]],
                bufs[c % 2],
                gsem,
            )

        def w(c):
            return pltpu.make_async_copy(
                bufs[c % 2], out_hbm.at[pl.ds(base + c * c_rows, c_rows)],
                wsem,
            )

        g(0).start()
        for c in range(chunks):
            g(c).wait()
            w(c).start()
            if c + 1 < chunks:
                if c >= 1:
                    w(c - 1).wait()
                g(c + 1).start()
        w(chunks - 2).wait()
        w(chunks - 1).wait()

    return gather


def _proj_body(x_ref, p_ref, w_ref, b_ref, o_ref):
    u = lax.bitcast_convert_type(x_ref[...], jnp.uint32)
    lo = lax.bitcast_convert_type(u.astype(jnp.uint16), jnp.bfloat16)
    hi = lax.bitcast_convert_type(
        (u >> 16).astype(jnp.uint16), jnp.bfloat16
    )
    v = p_ref[...]
    xh = jnp.where((v >> 18) % 2 == 1, hi, lo)
    half = jax.lax.broadcasted_iota(jnp.int32, u.shape, 1) // 64
    xsel = jnp.where(half == (v >> 19), xh, jnp.bfloat16(0.0))
    o_ref[...] = (
        jnp.dot(xsel, w_ref[...], preferred_element_type=jnp.float32)
        + b_ref[...]
    )


def _project(emb, parity, W2, b, block_rows=2560):
    n = emb.shape[0]
    h = W2.shape[1]
    return pl.pallas_call(
        _proj_body,
        grid=(n // block_rows,),
        in_specs=[
            pl.BlockSpec((block_rows, _PAIR), lambda i: (i, 0)),
            pl.BlockSpec((block_rows, 1), lambda i: (i, 0)),
            pl.BlockSpec((_PAIR, h), lambda i: (0, 0)),
            pl.BlockSpec((1, h), lambda i: (0, 0)),
        ],
        out_specs=pl.BlockSpec((block_rows, h), lambda i: (i, 0)),
        out_shape=jax.ShapeDtypeStruct((n, h), jnp.float32),
        compiler_params=pltpu.CompilerParams(
            dimension_semantics=("parallel",)
        ),
    )(emb, parity, W2, b.reshape(1, h))


def kernel(text_ids, table, W, b):
    batch, seq = text_ids.shape
    vocab, d = table.shape
    h = W.shape[1]
    idx = text_ids.T.reshape(-1)
    n = idx.shape[0]
    table2 = _pair_table(table.T)
    idx2 = idx & (_QUART - 1)
    parity = idx.reshape(n, 1)
    gather = _make_sc_gather(_QUART, n, chunks=4)
    emb2 = gather(table2, idx2)
    W2 = jnp.concatenate([W, W], axis=0).astype(jnp.bfloat16)
    out = _project(emb2, parity, W2, b)
    return out.reshape(seq, batch, h).transpose(1, 0, 2)
